# baseline (speedup 1.0000x reference)
import functools

import jax
import jax.numpy as jnp
from jax import lax
from jax.experimental import pallas as pl
from jax.experimental.pallas import tpu as pltpu
from jax.experimental.pallas import tpu_sc as plsc

_EMBED_DIM = 32
_SEQ_TILE = 2048


_ROW_PAD = 128


def _floor_gather_sc(floors, emb_table):
    n = floors.shape[0] // 2
    n_chunks = (n + 15) // 16
    last_base = n - 16
    mesh = plsc.VectorSubcoreMesh(core_axis_name="c", subcore_axis_name="s",
                                  num_cores=1)

    @functools.partial(
        pl.kernel,
        mesh=mesh,
        out_type=(jax.ShapeDtypeStruct((n * _EMBED_DIM,), jnp.float32),
                  jax.ShapeDtypeStruct((2, _ROW_PAD), jnp.float32)),
        scratch_types=[
            pltpu.VMEM((2, _EMBED_DIM), jnp.float32),
            pltpu.VMEM((2, _ROW_PAD), jnp.float32),
            pltpu.VMEM((16,), jnp.int32),
            pltpu.VMEM((16,), jnp.int32),
            pltpu.VMEM((16,), jnp.int32),
            pltpu.VMEM((16, _ROW_PAD), jnp.float32),
            pltpu.VMEM((16 * _EMBED_DIM,), jnp.float32),
            pltpu.SemaphoreType.DMA,
            pltpu.SemaphoreType.DMA,
        ],
    )
    def gather_kernel(floors_hbm, table_hbm, out_hbm, tpad_hbm,
                      tab_v, pad_v, src_v, tgt_v, idx_v, rows_v, fv_v,
                      sem1, sem2):
        wid = lax.axis_index("s") + lax.axis_index("c")

        @pl.when(wid < n_chunks)
        def _():
            base = jnp.minimum(wid * 16, last_base)
            cp1 = pltpu.async_copy(floors_hbm.at[pl.ds(base, 16)], src_v, sem1)
            cp2 = pltpu.async_copy(floors_hbm.at[pl.ds(n + base, 16)],
                                   tgt_v, sem2)
            pltpu.sync_copy(table_hbm, tab_v)
            for r in range(2):
                for c in range(_EMBED_DIM // 16):
                    pad_v[r, pl.ds(c * 16, 16)] = tab_v[r, pl.ds(c * 16, 16)]
            pltpu.sync_copy(pad_v, tpad_hbm)
            cp1.wait()
            cp2.wait()
            idx_v[...] = lax.bitwise_xor(
                lax.bitwise_xor(src_v[...], tgt_v[...]), 1)
            pltpu.async_copy(tpad_hbm.at[idx_v], rows_v, sem1).wait()
            for i in range(16):
                for c in range(_EMBED_DIM // 16):
                    fv_v[pl.ds(i * _EMBED_DIM + c * 16, 16)] = (
                        rows_v[i, pl.ds(c * 16, 16)])
            pltpu.sync_copy(
                fv_v, out_hbm.at[pl.ds(base * _EMBED_DIM, 16 * _EMBED_DIM)])

    fv, _ = gather_kernel(floors, emb_table)
    return fv


def _proj_body(x_ref, w_ref, fv_ref, b_ref, o_ref, bias_ref, *, d, hv):
    i = pl.program_id(0)
    j = pl.program_id(1)
    nb = bias_ref.shape[0]

    @pl.when((i == 0) & (j == 0))
    def _():
        w2 = w_ref[d:, :]
        for bb in range(nb):
            fvb = fv_ref[pl.ds(bb * hv, hv)].reshape(1, hv)
            bias_ref[pl.ds(bb, 1), :] = (
                jnp.dot(fvb, w2, preferred_element_type=jnp.float32)
                + b_ref[...])

    o_ref[...] = (jnp.dot(x_ref[0], w_ref[:d, :],
                          preferred_element_type=jnp.float32)
                  + bias_ref[pl.ds(i, 1), :])[None]


def kernel(word_encodings, src_floors, tgt_floors, emb_table, W, b):
    B, S, D = word_encodings.shape
    hist = src_floors.shape[1]
    hv = hist * _EMBED_DIM

    floors = jnp.concatenate(
        [src_floors.astype(jnp.int32).reshape(-1),
         tgt_floors.astype(jnp.int32).reshape(-1)], axis=0)
    fv_flat = _floor_gather_sc(floors, emb_table)

    grid = (B, S // _SEQ_TILE)
    out = pl.pallas_call(
        functools.partial(_proj_body, d=D, hv=hv),
        grid=grid,
        in_specs=[
            pl.BlockSpec((1, _SEQ_TILE, D), lambda i, j: (i, j, 0)),
            pl.BlockSpec((D + hv, D), lambda i, j: (0, 0)),
            pl.BlockSpec((B * hv,), lambda i, j: (0,)),
            pl.BlockSpec((1, D), lambda i, j: (0, 0)),
        ],
        out_specs=pl.BlockSpec((1, _SEQ_TILE, D), lambda i, j: (i, j, 0)),
        out_shape=jax.ShapeDtypeStruct((B, S, D), jnp.float32),
        scratch_shapes=[pltpu.VMEM((B, D), jnp.float32)],
        compiler_params=pltpu.CompilerParams(
            dimension_semantics=("arbitrary", "arbitrary"),
        ),
    )(word_encodings, W, fv_flat, b.reshape(1, D))
    return out


# speedup vs baseline: 1.0308x; 1.0308x over previous
import functools

import jax
import jax.numpy as jnp
from jax import lax
from jax.experimental import pallas as pl
from jax.experimental.pallas import tpu as pltpu
from jax.experimental.pallas import tpu_sc as plsc

_EMBED_DIM = 32
_SEQ_TILE = 2048


_ROW_PAD = 128


def _floor_gather_sc(floors, emb_table):
    n = floors.shape[0] // 2
    n_chunks = (n + 15) // 16
    last_base = n - 16
    mesh = plsc.VectorSubcoreMesh(core_axis_name="c", subcore_axis_name="s",
                                  num_cores=1)

    @functools.partial(
        pl.kernel,
        mesh=mesh,
        out_type=jax.ShapeDtypeStruct((n * _EMBED_DIM,), jnp.float32),
        scratch_types=[
            pltpu.VMEM((16,), jnp.int32),
            pltpu.VMEM((16,), jnp.int32),
            pltpu.VMEM((16,), jnp.int32),
            pltpu.VMEM((16, _ROW_PAD), jnp.float32),
            pltpu.VMEM((16 * _EMBED_DIM,), jnp.float32),
            pltpu.SemaphoreType.DMA,
            pltpu.SemaphoreType.DMA,
        ],
    )
    def gather_kernel(floors_hbm, table_hbm, out_hbm,
                      src_v, tgt_v, idx_v, rows_v, fv_v,
                      sem1, sem2):
        wid = lax.axis_index("s") + lax.axis_index("c")

        @pl.when(wid < n_chunks)
        def _():
            base = jnp.minimum(wid * 16, last_base)
            cp1 = pltpu.async_copy(floors_hbm.at[pl.ds(base, 16)], src_v, sem1)
            cp2 = pltpu.async_copy(floors_hbm.at[pl.ds(n + base, 16)],
                                   tgt_v, sem2)
            cp1.wait()
            cp2.wait()
            idx_v[...] = lax.bitwise_xor(
                lax.bitwise_xor(src_v[...], tgt_v[...]), 1)
            pltpu.async_copy(table_hbm.at[idx_v], rows_v, sem1).wait()
            for i in range(16):
                for c in range(_EMBED_DIM // 16):
                    fv_v[pl.ds(i * _EMBED_DIM + c * 16, 16)] = (
                        rows_v[i, pl.ds(c * 16, 16)])
            pltpu.sync_copy(
                fv_v, out_hbm.at[pl.ds(base * _EMBED_DIM, 16 * _EMBED_DIM)])

    return gather_kernel(floors, emb_table)


def _proj_body(x_ref, w_ref, fv_ref, b_ref, o_ref, bias_ref, *, d, hv):
    i = pl.program_id(0)
    j = pl.program_id(1)
    nb = bias_ref.shape[0]

    @pl.when((i == 0) & (j == 0))
    def _():
        w2 = w_ref[d:, :]
        for bb in range(nb):
            fvb = fv_ref[pl.ds(bb * hv, hv)].reshape(1, hv)
            bias_ref[pl.ds(bb, 1), :] = (
                jnp.dot(fvb, w2, preferred_element_type=jnp.float32)
                + b_ref[...])

    o_ref[...] = (jnp.dot(x_ref[0], w_ref[:d, :],
                          preferred_element_type=jnp.float32)
                  + bias_ref[pl.ds(i, 1), :])[None]


def kernel(word_encodings, src_floors, tgt_floors, emb_table, W, b):
    B, S, D = word_encodings.shape
    hist = src_floors.shape[1]
    hv = hist * _EMBED_DIM

    floors = jnp.concatenate(
        [src_floors.astype(jnp.int32).reshape(-1),
         tgt_floors.astype(jnp.int32).reshape(-1)], axis=0)
    fv_flat = _floor_gather_sc(
        floors, jnp.pad(emb_table, ((0, 0), (0, _ROW_PAD - _EMBED_DIM))))

    grid = (B, S // _SEQ_TILE)
    out = pl.pallas_call(
        functools.partial(_proj_body, d=D, hv=hv),
        grid=grid,
        in_specs=[
            pl.BlockSpec((1, _SEQ_TILE, D), lambda i, j: (i, j, 0)),
            pl.BlockSpec((D + hv, D), lambda i, j: (0, 0)),
            pl.BlockSpec((B * hv,), lambda i, j: (0,)),
            pl.BlockSpec((1, D), lambda i, j: (0, 0)),
        ],
        out_specs=pl.BlockSpec((1, _SEQ_TILE, D), lambda i, j: (i, j, 0)),
        out_shape=jax.ShapeDtypeStruct((B, S, D), jnp.float32),
        scratch_shapes=[pltpu.VMEM((B, D), jnp.float32)],
        compiler_params=pltpu.CompilerParams(
            dimension_semantics=("arbitrary", "arbitrary"),
        ),
    )(word_encodings, W, fv_flat, b.reshape(1, D))
    return out


# speedup vs baseline: 1.0444x; 1.0132x over previous
import functools

import jax
import jax.numpy as jnp
from jax import lax
from jax.experimental import pallas as pl
from jax.experimental.pallas import tpu as pltpu
from jax.experimental.pallas import tpu_sc as plsc

_EMBED_DIM = 32
_SEQ_TILE = 2048


_ROW_PAD = 128


def _floor_gather_sc(floors, emb_table):
    n = floors.shape[0] // 2
    n_chunks = (n + 15) // 16
    last_base = n - 16
    mesh = plsc.VectorSubcoreMesh(core_axis_name="c", subcore_axis_name="s",
                                  num_cores=1)

    @functools.partial(
        pl.kernel,
        mesh=mesh,
        out_type=jax.ShapeDtypeStruct((n * _EMBED_DIM,), jnp.float32),
        scratch_types=[
            pltpu.VMEM((16,), jnp.int32),
            pltpu.VMEM((16,), jnp.int32),
            pltpu.VMEM((16,), jnp.int32),
            pltpu.VMEM((16, _ROW_PAD), jnp.float32),
            pltpu.VMEM((16 * _EMBED_DIM,), jnp.float32),
            pltpu.SemaphoreType.DMA,
            pltpu.SemaphoreType.DMA,
        ],
    )
    def gather_kernel(floors_hbm, table_hbm, out_hbm,
                      src_v, tgt_v, idx_v, rows_v, fv_v,
                      sem1, sem2):
        wid = lax.axis_index("s") + lax.axis_index("c")

        @pl.when(wid < n_chunks)
        def _():
            base = jnp.minimum(wid * 16, last_base)
            cp1 = pltpu.async_copy(floors_hbm.at[pl.ds(base, 16)], src_v, sem1)
            cp2 = pltpu.async_copy(floors_hbm.at[pl.ds(n + base, 16)],
                                   tgt_v, sem2)
            cp1.wait()
            cp2.wait()
            idx_v[...] = lax.bitwise_xor(
                lax.bitwise_xor(src_v[...], tgt_v[...]), 1)
            pltpu.async_copy(table_hbm.at[idx_v], rows_v, sem1).wait()
            for i in range(16):
                for c in range(_EMBED_DIM // 16):
                    fv_v[pl.ds(i * _EMBED_DIM + c * 16, 16)] = (
                        rows_v[i, pl.ds(c * 16, 16)])
            pltpu.sync_copy(
                fv_v, out_hbm.at[pl.ds(base * _EMBED_DIM, 16 * _EMBED_DIM)])

    return gather_kernel(floors, emb_table)


def _prep_body(src_ref, tgt_ref, tab_ref, fl_ref, tp_ref):
    nb, h = src_ref.shape
    for b in range(nb):
        fl_ref[pl.ds(b * h, h)] = src_ref[b, :]
        fl_ref[pl.ds(nb * h + b * h, h)] = tgt_ref[b, :]
    tp_ref[...] = jnp.zeros(tp_ref.shape, jnp.float32)
    tp_ref[:, pl.ds(0, tab_ref.shape[1])] = tab_ref[...]


def _prep_inputs(src, tgt, table):
    nb, h = src.shape
    return pl.pallas_call(
        _prep_body,
        out_shape=(jax.ShapeDtypeStruct((2 * nb * h,), jnp.int32),
                   jax.ShapeDtypeStruct((2, _ROW_PAD), jnp.float32)),
    )(src, tgt, table)


def _proj_body(x_ref, w_ref, fv_ref, b_ref, o_ref, bias_ref, *, d, hv):
    i = pl.program_id(0)
    j = pl.program_id(1)
    nb = bias_ref.shape[0]

    @pl.when((i == 0) & (j == 0))
    def _():
        w2 = w_ref[d:, :]
        for bb in range(nb):
            fvb = fv_ref[pl.ds(bb * hv, hv)].reshape(1, hv)
            bias_ref[pl.ds(bb, 1), :] = (
                jnp.dot(fvb, w2, preferred_element_type=jnp.float32)
                + b_ref[...])

    o_ref[...] = (jnp.dot(x_ref[0], w_ref[:d, :],
                          preferred_element_type=jnp.float32)
                  + bias_ref[pl.ds(i, 1), :])[None]


def kernel(word_encodings, src_floors, tgt_floors, emb_table, W, b):
    B, S, D = word_encodings.shape
    hist = src_floors.shape[1]
    hv = hist * _EMBED_DIM

    floors, table_padded = _prep_inputs(
        src_floors.astype(jnp.int32), tgt_floors.astype(jnp.int32), emb_table)
    fv_flat = _floor_gather_sc(floors, table_padded)

    grid = (B, S // _SEQ_TILE)
    out = pl.pallas_call(
        functools.partial(_proj_body, d=D, hv=hv),
        grid=grid,
        in_specs=[
            pl.BlockSpec((1, _SEQ_TILE, D), lambda i, j: (i, j, 0)),
            pl.BlockSpec((D + hv, D), lambda i, j: (0, 0)),
            pl.BlockSpec((B * hv,), lambda i, j: (0,)),
            pl.BlockSpec((1, D), lambda i, j: (0, 0)),
        ],
        out_specs=pl.BlockSpec((1, _SEQ_TILE, D), lambda i, j: (i, j, 0)),
        out_shape=jax.ShapeDtypeStruct((B, S, D), jnp.float32),
        scratch_shapes=[pltpu.VMEM((B, D), jnp.float32)],
        compiler_params=pltpu.CompilerParams(
            dimension_semantics=("arbitrary", "arbitrary"),
        ),
    )(word_encodings, W, fv_flat, b.reshape(1, D))
    return out
